# baseline (speedup 1.0000x reference)
import functools

import jax
import jax.numpy as jnp
from jax import lax
from jax.experimental import pallas as pl
from jax.experimental.pallas import tpu as pltpu
from jax.experimental.pallas import tpu_sc as plsc

_K = 4
_S = 256
_C = 392
_RO = 4 * 96
_R = 4
_TN = _R * 112
_N = 12544
_NW = 32
_TPW = 392
_TCH = 8


def _mm(a, b):
    return lax.dot_general(a, b, (((1,), (0,)), ((), ())),
                           preferred_element_type=jnp.float32)


def _mmT(a, b):
    return lax.dot_general(a, b, (((0,), (0,)), ((), ())),
                           preferred_element_type=jnp.float32)


def _zf_body(xs_ref, wz_ref, zf_ref):
    zf_ref[0] = _mm(xs_ref[0], wz_ref[...])


def _score_body(x_ref, xsij_ref, ids_ref):
    f32 = jnp.float32
    b0 = pl.program_id(0)
    i0 = pl.program_id(1)
    xb = x_ref[0]
    r224 = lax.broadcasted_iota(jnp.int32, (224, 112), 0)
    c224 = lax.broadcasted_iota(jnp.int32, (224, 112), 1)
    Pe = (r224 == 2 * c224).astype(f32)
    Po = (r224 == 2 * c224 + 1).astype(f32)
    xf = xb.reshape(96 * 2 * _R, 224)
    xje = _mm(xf, Pe).reshape(96, 2 * _R, 112)
    xjo = _mm(xf, Po).reshape(96, 2 * _R, 112)
    w112 = lax.broadcasted_iota(jnp.int32, (1, 112), 1).astype(f32)
    sn2 = jnp.sum(xsij_ref[0] * xsij_ref[0], axis=(0, 2)).reshape(_S, 1)
    iota_s = lax.broadcasted_iota(jnp.int32, (_S, 112), 0)
    for hu in range(_R):
        innerT = jnp.zeros((_S, 112), f32)
        for i in range(2):
            rp = (i0 * 2 * _R + 2 * hu + i).astype(f32)
            for j in range(2):
                colpix = 2.0 * w112 + float(j)
                nrm = jnp.maximum(jnp.sqrt(rp * rp + colpix * colpix), 1e-12)
                xj = xje if j == 0 else xjo
                xij = jnp.concatenate(
                    [xj[:, 2 * hu + i, :], rp / nrm, colpix / nrm],
                    axis=0)
                innerT = innerT + _mm(xsij_ref[0, 2 * i + j], xij)
        neg = 2.0 * innerT - sn2
        for k in range(_K):
            m = jnp.max(neg, axis=0, keepdims=True)
            hit = neg == m
            idx = jnp.min(jnp.where(hit, iota_s, _S), axis=0, keepdims=True)
            sel = iota_s == idx
            gid = b0 * (_S * _K) + idx * _K + k
            ids_ref[0, k, 0, pl.ds(hu, 1), :] = gid
            neg = jnp.where(sel, -jnp.inf, neg)


def _inter_body(scv_ref, b2_ref, *rest, batch=0, has_prev=False):
    out_ref = rest[-1]
    f32 = jnp.float32
    r112 = lax.broadcasted_iota(jnp.int32, (112, 224), 0)
    c112 = lax.broadcasted_iota(jnp.int32, (112, 224), 1)
    Qe = (c112 == 2 * r112).astype(f32)
    Qo = (c112 == 2 * r112 + 1).astype(f32)
    blk = scv_ref[0, 0] + b2_ref[...]
    for hu in range(_R):
        piece = blk[hu * 112:(hu + 1) * 112, :]
        for i in range(2):
            row = (_mmT(piece[:, (2 * i) * 96:(2 * i + 1) * 96], Qe)
                   + _mmT(piece[:, (2 * i + 1) * 96:(2 * i + 2) * 96], Qo))
            out_ref[0, :, 2 * hu + i, :] = row


def _sc_gather(zrow, idsf, n_tok):
    mesh = plsc.VectorSubcoreMesh(core_axis_name="c", subcore_axis_name="s")
    nch = _TPW // _TCH

    @functools.partial(
        pl.kernel, mesh=mesh,
        out_type=jax.ShapeDtypeStruct((n_tok, _RO), jnp.float32),
        scratch_types=(
            [pltpu.VMEM((_TPW,), jnp.int32) for _ in range(_K)]
            + [pltpu.VMEM((_TCH, _RO), jnp.float32) for _ in range(2 * _K)]
            + [pltpu.SemaphoreType.DMA, pltpu.SemaphoreType.DMA,
               pltpu.SemaphoreType.DMA]
        ),
    )
    def k(zrow_hbm, ids_hbm, out_hbm, *refs):
        ivf = refs[0:4]
        gvs = (refs[4:8], refs[8:12])
        sems = refs[12:14]
        semo = refs[14]
        wid = lax.axis_index("s") * 2 + lax.axis_index("c")
        tok0 = wid * _TPW
        for kk in range(_K):
            pltpu.sync_copy(ids_hbm.at[pl.ds(kk * _N + tok0, _TPW)],
                            ivf[kk])

        def fire(c, s):
            for kk in range(_K):
                pltpu.async_copy(
                    zrow_hbm.at[ivf[kk].at[pl.ds(c * _TCH, _TCH)]],
                    gvs[s][kk], sems[s])

        def proc(c, s):
            for kk in range(_K):
                pltpu.make_async_copy(
                    zrow_hbm.at[pl.ds(0, _TCH)], gvs[s][kk], sems[s]).wait()
            g0, g1, g2, g3 = gvs[s]

            @pl.when(c >= 2)
            def _():
                pltpu.make_async_copy(
                    g0, out_hbm.at[pl.ds(tok0, _TCH)], semo).wait()

            def rows(r, c2):
                for cv in range(_RO // 16):
                    sl = pl.ds(cv * 16, 16)
                    g0[r, sl] = (g0[r, sl] + g1[r, sl]
                                 + g2[r, sl] + g3[r, sl])
                return c2
            lax.fori_loop(0, _TCH, rows, 0)
            pltpu.async_copy(g0, out_hbm.at[pl.ds(tok0 + c * _TCH, _TCH)],
                             semo)

        fire(0, 0)

        def body(cc, carry):
            c0 = 2 * cc
            fire(c0 + 1, 1)
            proc(c0, 0)

            @pl.when(c0 + 2 < nch)
            def _():
                fire(c0 + 2, 0)
            proc(c0 + 1, 1)
            return carry

        lax.fori_loop(0, nch // 2, body, 0)
        if nch % 2:
            proc(nch - 1, 0)
        for _t in range(2):
            pltpu.make_async_copy(
                gvs[_t][0], out_hbm.at[pl.ds(tok0, _TCH)], semo).wait()

    return k(zrow, idsf)


def kernel(x, Wc, bc, Wp, bp):
    B, Cin, H, W = x.shape
    Hu, Wu = H // 2, W // 2
    f32 = jnp.float32
    ind = jnp.round(jnp.linspace(0, Hu - 1, 16)).astype(jnp.int32)
    xs4 = jnp.stack([x[:, :, 2 * ind + i, :][:, :, :, 2 * ind + j]
                     for i in range(2) for j in range(2)], axis=1)
    xg = jnp.arange(H, dtype=f32)
    coord_r = jnp.broadcast_to(xg[:, None], (H, W))
    coord_c = jnp.broadcast_to(xg[None, :], (H, W))
    nrm = jnp.maximum(jnp.sqrt(coord_r**2 + coord_c**2), 1e-12)
    cr, cc = coord_r / nrm, coord_c / nrm
    cs4 = jnp.stack([jnp.stack([cr[2 * ind + i, :][:, 2 * ind + j],
                                cc[2 * ind + i, :][:, 2 * ind + j]])
                     for i in range(2) for j in range(2)], axis=0)
    cs4 = jnp.broadcast_to(cs4[None], (B, 4, 2, 16, 16))
    xsij = jnp.concatenate([xs4, cs4], axis=2)
    xsij = xsij.reshape(B, 4, 98, _S)
    xsijT = xsij.transpose(0, 1, 3, 2)
    xs = xsij.transpose(0, 3, 2, 1).reshape(B, _S, 98 * 4)
    Wc4 = Wc.reshape(Cin + 2, 4, _C, _K)
    Wz = jnp.einsum('op,prck->ckro', Wp, Wc4).reshape(_C, _K * _RO)
    b2 = (jnp.einsum('op,pr->ro', Wp, bc.reshape(Cin + 2, 4))
          + bp[None, :]).reshape(1, _RO)

    z2 = pl.pallas_call(
        _zf_body,
        grid=(B,),
        in_specs=[
            pl.BlockSpec((1, _S, _C), lambda b: (b, 0, 0)),
            pl.BlockSpec((_C, _K * _RO), lambda b: (0, 0)),
        ],
        out_specs=pl.BlockSpec((1, _S, _K * _RO), lambda b: (b, 0, 0)),
        out_shape=jax.ShapeDtypeStruct((B, _S, _K * _RO), f32),
    )(xs, Wz)
    zrow = z2.reshape(B * _S * _K, _RO)

    scs = []
    for b in range(B):
        ids_b = pl.pallas_call(
            _score_body,
            grid=(1, Hu // _R),
            in_specs=[
                pl.BlockSpec((1, Cin, 2 * _R, W), lambda bb, i: (bb, 0, i, 0)),
                pl.BlockSpec((1, 4, _S, 98), lambda bb, i: (bb, 0, 0, 0)),
            ],
            out_specs=pl.BlockSpec((1, _K, 1, _R, 112),
                                   lambda bb, i: (bb, 0, i, 0, 0)),
            out_shape=jax.ShapeDtypeStruct((1, _K, Hu // _R, _R, 112),
                                           jnp.int32),
        )(x[b:b + 1], xsijT[b:b + 1])
        sc_b = _sc_gather(zrow[b * _S * _K:(b + 1) * _S * _K],
                          ids_b.reshape(_K * _N), _N)
        scs.append(sc_b.reshape(1, Hu // _R, _TN, _RO))

    out = None
    for b in range(B):
        alias = {} if out is None else {2: 0}
        args = (scs[b], b2) if out is None else (scs[b], b2, out)
        in_specs = [
            pl.BlockSpec((1, 1, _TN, _RO), lambda i, bb=b: (0, i, 0, 0)),
            pl.BlockSpec((1, _RO), lambda i: (0, 0)),
        ]
        if out is not None:
            in_specs.append(pl.BlockSpec(memory_space=pl.ANY))
        out = pl.pallas_call(
            functools.partial(_inter_body, batch=b, has_prev=out is not None),
            grid=(Hu // _R,),
            in_specs=in_specs,
            out_specs=pl.BlockSpec((1, 96, 2 * _R, W),
                                   lambda i, bb=b: (bb, 0, i, 0)),
            out_shape=jax.ShapeDtypeStruct((B, 96, H, W), f32),
            input_output_aliases=alias,
        )(*args)
    return out


# speedup vs baseline: 1.4379x; 1.4379x over previous
import functools

import jax
import jax.numpy as jnp
from jax import lax
from jax.experimental import pallas as pl
from jax.experimental.pallas import tpu as pltpu
from jax.experimental.pallas import tpu_sc as plsc

_K = 4
_S = 256
_C = 392
_RO = 4 * 96
_R = 4
_TN = _R * 112
_NT = 28
_T1 = 12
_N = 12544
_NW = 32
_TCH = 8


def _mm(a, b):
    return lax.dot_general(a, b, (((1,), (0,)), ((), ())),
                           preferred_element_type=jnp.float32)


def _mmT(a, b):
    return lax.dot_general(a, b, (((0,), (0,)), ((), ())),
                           preferred_element_type=jnp.float32)


def _zf_body(xs_ref, wz_ref, zf_ref):
    zf_ref[0] = _mm(xs_ref[0], wz_ref[...])


def _deinterleave(xb):
    f32 = jnp.float32
    r224 = lax.broadcasted_iota(jnp.int32, (224, 112), 0)
    c224 = lax.broadcasted_iota(jnp.int32, (224, 112), 1)
    Pe = (r224 == 2 * c224).astype(f32)
    Po = (r224 == 2 * c224 + 1).astype(f32)
    xf = xb.reshape(96 * 2 * _R, 224)
    xje = _mm(xf, Pe).reshape(96, 2 * _R, 112)
    xjo = _mm(xf, Po).reshape(96, 2 * _R, 112)
    return xje, xjo


def _qmats():
    f32 = jnp.float32
    r112 = lax.broadcasted_iota(jnp.int32, (112, 224), 0)
    c112 = lax.broadcasted_iota(jnp.int32, (112, 224), 1)
    Qe = (c112 == 2 * r112).astype(f32)
    Qo = (c112 == 2 * r112 + 1).astype(f32)
    return Qe, Qo


def _scores_hu(xje, xjo, xsij_ref, hu, rowbase):
    f32 = jnp.float32
    w112 = lax.broadcasted_iota(jnp.int32, (1, 112), 1).astype(f32)
    innerT = jnp.zeros((_S, 112), f32)
    for i in range(2):
        rp = (rowbase + 2 * hu + i).astype(f32)
        for j in range(2):
            colpix = 2.0 * w112 + float(j)
            nrm = jnp.maximum(jnp.sqrt(rp * rp + colpix * colpix), 1e-12)
            xj = xje if j == 0 else xjo
            xij = jnp.concatenate(
                [xj[:, 2 * hu + i, :], rp / nrm, colpix / nrm], axis=0)
            innerT = innerT + _mm(xsij_ref[0, 2 * i + j], xij)
    return innerT


def _fused_body(x_ref, xsij_ref, z2_ref, b2_ref, *rest):
    f32 = jnp.float32
    out_ref = rest[-1]
    i0 = pl.program_id(1)
    xje, xjo = _deinterleave(x_ref[0])
    Qe, Qo = _qmats()
    sn2 = jnp.sum(xsij_ref[0] * xsij_ref[0], axis=(0, 2)).reshape(_S, 1)
    iota_s = lax.broadcasted_iota(jnp.int32, (_S, 112), 0)
    z2 = z2_ref[0]
    b2 = b2_ref[...]
    for hu in range(_R):
        innerT = _scores_hu(xje, xjo, xsij_ref, hu, i0 * 2 * _R)
        neg = 2.0 * innerT - sn2
        accT = b2
        for k in range(_K):
            m = jnp.max(neg, axis=0, keepdims=True)
            hit = neg == m
            idx = jnp.min(jnp.where(hit, iota_s, _S), axis=0, keepdims=True)
            sel = iota_s == idx
            accT = accT + _mmT(z2[:, k * _RO:(k + 1) * _RO], sel.astype(f32))
            neg = jnp.where(sel, -jnp.inf, neg)
        for i in range(2):
            row = (_mm(accT[(2 * i) * 96:(2 * i + 1) * 96], Qe)
                   + _mm(accT[(2 * i + 1) * 96:(2 * i + 2) * 96], Qo))
            out_ref[0, :, 2 * hu + i, :] = row


def _score_body(x_ref, xsij_ref, ids_ref):
    i0 = pl.program_id(1) + _T1
    xje, xjo = _deinterleave(x_ref[0])
    sn2 = jnp.sum(xsij_ref[0] * xsij_ref[0], axis=(0, 2)).reshape(_S, 1)
    iota_s = lax.broadcasted_iota(jnp.int32, (_S, 112), 0)
    for hu in range(_R):
        innerT = _scores_hu(xje, xjo, xsij_ref, hu, i0 * 2 * _R)
        neg = 2.0 * innerT - sn2
        for k in range(_K):
            m = jnp.max(neg, axis=0, keepdims=True)
            hit = neg == m
            idx = jnp.min(jnp.where(hit, iota_s, _S), axis=0, keepdims=True)
            sel = iota_s == idx
            ids_ref[0, k, 0, pl.ds(hu, 1), :] = idx * _K + k
            neg = jnp.where(sel, -jnp.inf, neg)


def _inter_body(scv_ref, b2_ref, *rest):
    out_ref = rest[-1]
    Qe, Qo = _qmats()
    blk = scv_ref[0, 0] + b2_ref[...]
    for hu in range(_R):
        piece = blk[hu * 112:(hu + 1) * 112, :]
        for i in range(2):
            row = (_mmT(piece[:, (2 * i) * 96:(2 * i + 1) * 96], Qe)
                   + _mmT(piece[:, (2 * i + 1) * 96:(2 * i + 2) * 96], Qo))
            out_ref[0, :, 2 * hu + i, :] = row


def _sc_gather(zrow, idsf, n_tok):
    mesh = plsc.VectorSubcoreMesh(core_axis_name="c", subcore_axis_name="s")
    tpw = n_tok // _NW
    nch = tpw // _TCH

    @functools.partial(
        pl.kernel, mesh=mesh,
        out_type=jax.ShapeDtypeStruct((n_tok, _RO), jnp.float32),
        scratch_types=(
            [pltpu.VMEM((tpw,), jnp.int32) for _ in range(_K)]
            + [pltpu.VMEM((_TCH, _RO), jnp.float32) for _ in range(2 * _K)]
            + [pltpu.SemaphoreType.DMA, pltpu.SemaphoreType.DMA,
               pltpu.SemaphoreType.DMA]
        ),
    )
    def k(zrow_hbm, ids_hbm, out_hbm, *refs):
        ivf = refs[0:4]
        gvs = (refs[4:8], refs[8:12])
        sems = refs[12:14]
        semo = refs[14]
        wid = lax.axis_index("s") * 2 + lax.axis_index("c")
        tok0 = wid * tpw
        for kk in range(_K):
            pltpu.sync_copy(ids_hbm.at[pl.ds(kk * n_tok + tok0, tpw)],
                            ivf[kk])

        def fire(c, s):
            for kk in range(_K):
                pltpu.async_copy(
                    zrow_hbm.at[ivf[kk].at[pl.ds(c * _TCH, _TCH)]],
                    gvs[s][kk], sems[s])

        def proc(c, s):
            for kk in range(_K):
                pltpu.make_async_copy(
                    zrow_hbm.at[pl.ds(0, _TCH)], gvs[s][kk], sems[s]).wait()
            g0, g1, g2, g3 = gvs[s]

            @pl.when(c >= 2)
            def _():
                pltpu.make_async_copy(
                    g0, out_hbm.at[pl.ds(tok0, _TCH)], semo).wait()

            def rows(r, c2):
                for cv in range(_RO // 16):
                    sl = pl.ds(cv * 16, 16)
                    g0[r, sl] = (g0[r, sl] + g1[r, sl]
                                 + g2[r, sl] + g3[r, sl])
                return c2
            lax.fori_loop(0, _TCH, rows, 0)
            pltpu.async_copy(g0, out_hbm.at[pl.ds(tok0 + c * _TCH, _TCH)],
                             semo)

        fire(0, 0)

        def body(cc, carry):
            c0 = 2 * cc
            fire(c0 + 1, 1)
            proc(c0, 0)

            @pl.when(c0 + 2 < nch)
            def _():
                fire(c0 + 2, 0)
            proc(c0 + 1, 1)
            return carry

        lax.fori_loop(0, nch // 2, body, 0)
        if nch % 2:
            proc(nch - 1, 0)
        for _t in range(2):
            pltpu.make_async_copy(
                gvs[_t][0], out_hbm.at[pl.ds(tok0, _TCH)], semo).wait()

    return k(zrow, idsf)


def kernel(x, Wc, bc, Wp, bp):
    B, Cin, H, W = x.shape
    Hu, Wu = H // 2, W // 2
    f32 = jnp.float32
    t2 = _NT - _T1
    nsc = t2 * _TN
    ind = jnp.round(jnp.linspace(0, Hu - 1, 16)).astype(jnp.int32)
    xs4 = jnp.stack([x[:, :, 2 * ind + i, :][:, :, :, 2 * ind + j]
                     for i in range(2) for j in range(2)], axis=1)
    xg = jnp.arange(H, dtype=f32)
    coord_r = jnp.broadcast_to(xg[:, None], (H, W))
    coord_c = jnp.broadcast_to(xg[None, :], (H, W))
    nrm = jnp.maximum(jnp.sqrt(coord_r**2 + coord_c**2), 1e-12)
    cr, cc = coord_r / nrm, coord_c / nrm
    cs4 = jnp.stack([jnp.stack([cr[2 * ind + i, :][:, 2 * ind + j],
                                cc[2 * ind + i, :][:, 2 * ind + j]])
                     for i in range(2) for j in range(2)], axis=0)
    cs4 = jnp.broadcast_to(cs4[None], (B, 4, 2, 16, 16))
    xsij = jnp.concatenate([xs4, cs4], axis=2)
    xsij = xsij.reshape(B, 4, 98, _S)
    xsijT = xsij.transpose(0, 1, 3, 2)
    xs = xsij.transpose(0, 3, 2, 1).reshape(B, _S, 98 * 4)
    Wc4 = Wc.reshape(Cin + 2, 4, _C, _K)
    Wz = jnp.einsum('op,prck->ckro', Wp, Wc4).reshape(_C, _K * _RO)
    b2r = (jnp.einsum('op,pr->ro', Wp, bc.reshape(Cin + 2, 4))
           + bp[None, :]).reshape(1, _RO)
    b2c = b2r.reshape(_RO, 1)

    z2 = pl.pallas_call(
        _zf_body,
        grid=(B,),
        in_specs=[
            pl.BlockSpec((1, _S, _C), lambda b: (b, 0, 0)),
            pl.BlockSpec((_C, _K * _RO), lambda b: (0, 0)),
        ],
        out_specs=pl.BlockSpec((1, _S, _K * _RO), lambda b: (b, 0, 0)),
        out_shape=jax.ShapeDtypeStruct((B, _S, _K * _RO), f32),
    )(xs, Wz)

    scvs = []
    for b in range(B):
        ids_b = pl.pallas_call(
            _score_body,
            grid=(1, t2),
            in_specs=[
                pl.BlockSpec((1, Cin, 2 * _R, W),
                             lambda bb, i: (bb, 0, i + _T1, 0)),
                pl.BlockSpec((1, 4, _S, 98), lambda bb, i: (bb, 0, 0, 0)),
            ],
            out_specs=pl.BlockSpec((1, _K, 1, _R, 112),
                                   lambda bb, i: (bb, 0, i, 0, 0)),
            out_shape=jax.ShapeDtypeStruct((1, _K, t2, _R, 112), jnp.int32),
        )(x[b:b + 1], xsijT[b:b + 1])
        zrow_b = z2[b].reshape(_S * _K, _RO)
        sc_b = _sc_gather(zrow_b, ids_b.reshape(_K * nsc), nsc)
        scvs.append(sc_b.reshape(1, t2, _TN, _RO))

    out = None
    for b in range(B):
        in_specs = [
            pl.BlockSpec((1, Cin, 2 * _R, W), lambda bb, i: (bb, 0, i, 0)),
            pl.BlockSpec((1, 4, _S, 98), lambda bb, i: (bb, 0, 0, 0)),
            pl.BlockSpec((1, _S, _K * _RO), lambda bb, i: (bb, 0, 0)),
            pl.BlockSpec((_RO, 1), lambda bb, i: (0, 0)),
        ]
        args = [x[b:b + 1], xsijT[b:b + 1], z2[b:b + 1], b2c]
        alias = {}
        if out is not None:
            in_specs.append(pl.BlockSpec(memory_space=pl.ANY))
            args.append(out)
            alias = {4: 0}
        out = pl.pallas_call(
            _fused_body,
            grid=(1, _T1),
            in_specs=in_specs,
            out_specs=pl.BlockSpec((1, 96, 2 * _R, W),
                                   lambda bb, i, b_=b: (b_, 0, i, 0)),
            out_shape=jax.ShapeDtypeStruct((B, 96, H, W), f32),
            input_output_aliases=alias,
        )(*args)

    for b in range(B):
        out = pl.pallas_call(
            _inter_body,
            grid=(t2,),
            in_specs=[
                pl.BlockSpec((1, 1, _TN, _RO), lambda i: (0, i, 0, 0)),
                pl.BlockSpec((1, _RO), lambda i: (0, 0)),
                pl.BlockSpec(memory_space=pl.ANY),
            ],
            out_specs=pl.BlockSpec((1, 96, 2 * _R, W),
                                   lambda i, b_=b: (b_, 0, i + _T1, 0)),
            out_shape=jax.ShapeDtypeStruct((B, 96, H, W), f32),
            input_output_aliases={2: 0},
        )(scvs[b], b2r, out)
    return out


# speedup vs baseline: 1.4381x; 1.0002x over previous
import functools

import jax
import jax.numpy as jnp
from jax import lax
from jax.experimental import pallas as pl
from jax.experimental.pallas import tpu as pltpu
from jax.experimental.pallas import tpu_sc as plsc

_K = 4
_S = 256
_C = 392
_RO = 4 * 96
_R = 4
_TN = _R * 112
_NT = 28
_T1 = 12
_N = 12544
_NW = 32
_TCH = 8


def _mm(a, b):
    return lax.dot_general(a, b, (((1,), (0,)), ((), ())),
                           preferred_element_type=jnp.float32)


def _mmT(a, b):
    return lax.dot_general(a, b, (((0,), (0,)), ((), ())),
                           preferred_element_type=jnp.float32)


def _zf_body(xs_ref, wz_ref, zf_ref):
    zf_ref[0] = _mm(xs_ref[0], wz_ref[...])


def _deinterleave(xb):
    f32 = jnp.float32
    r224 = lax.broadcasted_iota(jnp.int32, (224, 112), 0)
    c224 = lax.broadcasted_iota(jnp.int32, (224, 112), 1)
    Pe = (r224 == 2 * c224).astype(f32)
    Po = (r224 == 2 * c224 + 1).astype(f32)
    xf = xb.reshape(96 * 2 * _R, 224)
    xje = _mm(xf, Pe).reshape(96, 2 * _R, 112)
    xjo = _mm(xf, Po).reshape(96, 2 * _R, 112)
    return xje, xjo


def _qmats():
    f32 = jnp.float32
    r112 = lax.broadcasted_iota(jnp.int32, (112, 224), 0)
    c112 = lax.broadcasted_iota(jnp.int32, (112, 224), 1)
    Qe = (c112 == 2 * r112).astype(f32)
    Qo = (c112 == 2 * r112 + 1).astype(f32)
    return Qe, Qo


def _scores_hu(xje, xjo, xsij_ref, hu, rowbase):
    f32 = jnp.float32
    w112 = lax.broadcasted_iota(jnp.int32, (1, 112), 1).astype(f32)
    innerT = jnp.zeros((_S, 112), f32)
    for i in range(2):
        rp = (rowbase + 2 * hu + i).astype(f32)
        for j in range(2):
            colpix = 2.0 * w112 + float(j)
            nrm = jnp.maximum(jnp.sqrt(rp * rp + colpix * colpix), 1e-12)
            xj = xje if j == 0 else xjo
            xij = jnp.concatenate(
                [xj[:, 2 * hu + i, :], rp / nrm, colpix / nrm], axis=0)
            innerT = innerT + _mm(xsij_ref[0, 2 * i + j], xij)
    return innerT


def _fused_body(x_ref, xsij_ref, z2_ref, b2_ref, *rest):
    f32 = jnp.float32
    out_ref = rest[-1]
    i0 = pl.program_id(1)
    xje, xjo = _deinterleave(x_ref[0])
    Qe, Qo = _qmats()
    sn2 = jnp.sum(xsij_ref[0] * xsij_ref[0], axis=(0, 2)).reshape(_S, 1)
    iota_s = lax.broadcasted_iota(jnp.int32, (_S, 112), 0)
    z2 = z2_ref[0]
    b2 = b2_ref[...]
    for hu in range(_R):
        innerT = _scores_hu(xje, xjo, xsij_ref, hu, i0 * 2 * _R)
        neg = 2.0 * innerT - sn2
        accT = b2
        for k in range(_K):
            m = jnp.max(neg, axis=0, keepdims=True)
            hit = neg == m
            idx = jnp.min(jnp.where(hit, iota_s, _S), axis=0, keepdims=True)
            sel = iota_s == idx
            accT = accT + _mmT(z2[:, k * _RO:(k + 1) * _RO], sel.astype(f32))
            neg = jnp.where(sel, -jnp.inf, neg)
        for i in range(2):
            row = (_mm(accT[(2 * i) * 96:(2 * i + 1) * 96], Qe)
                   + _mm(accT[(2 * i + 1) * 96:(2 * i + 2) * 96], Qo))
            out_ref[0, :, 2 * hu + i, :] = row


def _score_body(x_ref, xsij_ref, ids_ref):
    i0 = pl.program_id(1) + _T1
    xje, xjo = _deinterleave(x_ref[0])
    sn2 = jnp.sum(xsij_ref[0] * xsij_ref[0], axis=(0, 2)).reshape(_S, 1)
    iota_s = lax.broadcasted_iota(jnp.int32, (_S, 112), 0)
    for hu in range(_R):
        innerT = _scores_hu(xje, xjo, xsij_ref, hu, i0 * 2 * _R)
        neg = 2.0 * innerT - sn2
        for k in range(_K):
            m = jnp.max(neg, axis=0, keepdims=True)
            hit = neg == m
            idx = jnp.min(jnp.where(hit, iota_s, _S), axis=0, keepdims=True)
            sel = iota_s == idx
            ids_ref[0, k, 0, pl.ds(hu, 1), :] = idx * _K + k
            neg = jnp.where(sel, -jnp.inf, neg)


def _inter_body(scv_ref, b2_ref, *rest):
    out_ref = rest[-1]
    Qe, Qo = _qmats()
    blk = scv_ref[0, 0] + b2_ref[...]
    for hu in range(_R):
        piece = blk[hu * 112:(hu + 1) * 112, :]
        for i in range(2):
            row = (_mmT(piece[:, (2 * i) * 96:(2 * i + 1) * 96], Qe)
                   + _mmT(piece[:, (2 * i + 1) * 96:(2 * i + 2) * 96], Qo))
            out_ref[0, :, 2 * hu + i, :] = row


def _sc_gather(zrow, idsf, n_tok):
    mesh = plsc.VectorSubcoreMesh(core_axis_name="c", subcore_axis_name="s")
    tpw = n_tok // _NW
    nch = tpw // _TCH

    @functools.partial(
        pl.kernel, mesh=mesh,
        out_type=jax.ShapeDtypeStruct((n_tok, _RO), jnp.float32),
        scratch_types=(
            [pltpu.VMEM((tpw,), jnp.int32) for _ in range(_K)]
            + [pltpu.VMEM((_TCH, _RO), jnp.float32) for _ in range(2 * _K)]
            + [pltpu.SemaphoreType.DMA, pltpu.SemaphoreType.DMA,
               pltpu.SemaphoreType.DMA, pltpu.SemaphoreType.DMA]
        ),
    )
    def k(zrow_hbm, ids_hbm, out_hbm, *refs):
        ivf = refs[0:4]
        gvs = (refs[4:8], refs[8:12])
        sems = refs[12:14]
        semos = refs[14:16]
        wid = lax.axis_index("s") * 2 + lax.axis_index("c")
        tok0 = wid * tpw
        for kk in range(_K):
            pltpu.sync_copy(ids_hbm.at[pl.ds(kk * n_tok + tok0, tpw)],
                            ivf[kk])

        def fire(c, s):
            for kk in range(_K):
                pltpu.async_copy(
                    zrow_hbm.at[ivf[kk].at[pl.ds(c * _TCH, _TCH)]],
                    gvs[s][kk], sems[s])

        def proc(c, s):
            for kk in range(_K):
                pltpu.make_async_copy(
                    zrow_hbm.at[pl.ds(0, _TCH)], gvs[s][kk], sems[s]).wait()
            g0, g1, g2, g3 = gvs[s]

            @pl.when(c >= 2)
            def _():
                pltpu.make_async_copy(
                    g0, out_hbm.at[pl.ds(tok0, _TCH)], semos[s]).wait()

            def rows(r, c2):
                for cv in range(_RO // 16):
                    sl = pl.ds(cv * 16, 16)
                    g0[r, sl] = (g0[r, sl] + g1[r, sl]
                                 + g2[r, sl] + g3[r, sl])
                return c2
            lax.fori_loop(0, _TCH, rows, 0)
            pltpu.async_copy(g0, out_hbm.at[pl.ds(tok0 + c * _TCH, _TCH)],
                             semos[s])

        fire(0, 0)

        def body(cc, carry):
            c0 = 2 * cc
            fire(c0 + 1, 1)
            proc(c0, 0)

            @pl.when(c0 + 2 < nch)
            def _():
                fire(c0 + 2, 0)
            proc(c0 + 1, 1)
            return carry

        lax.fori_loop(0, nch // 2, body, 0)
        if nch % 2:
            proc(nch - 1, 0)
        for _t in range(2):
            pltpu.make_async_copy(
                gvs[_t][0], out_hbm.at[pl.ds(tok0, _TCH)], semos[_t]).wait()

    return k(zrow, idsf)


def kernel(x, Wc, bc, Wp, bp):
    B, Cin, H, W = x.shape
    Hu, Wu = H // 2, W // 2
    f32 = jnp.float32
    t2 = _NT - _T1
    nsc = t2 * _TN
    ind = jnp.round(jnp.linspace(0, Hu - 1, 16)).astype(jnp.int32)
    xs4 = jnp.stack([x[:, :, 2 * ind + i, :][:, :, :, 2 * ind + j]
                     for i in range(2) for j in range(2)], axis=1)
    xg = jnp.arange(H, dtype=f32)
    coord_r = jnp.broadcast_to(xg[:, None], (H, W))
    coord_c = jnp.broadcast_to(xg[None, :], (H, W))
    nrm = jnp.maximum(jnp.sqrt(coord_r**2 + coord_c**2), 1e-12)
    cr, cc = coord_r / nrm, coord_c / nrm
    cs4 = jnp.stack([jnp.stack([cr[2 * ind + i, :][:, 2 * ind + j],
                                cc[2 * ind + i, :][:, 2 * ind + j]])
                     for i in range(2) for j in range(2)], axis=0)
    cs4 = jnp.broadcast_to(cs4[None], (B, 4, 2, 16, 16))
    xsij = jnp.concatenate([xs4, cs4], axis=2)
    xsij = xsij.reshape(B, 4, 98, _S)
    xsijT = xsij.transpose(0, 1, 3, 2)
    xs = xsij.transpose(0, 3, 2, 1).reshape(B, _S, 98 * 4)
    Wc4 = Wc.reshape(Cin + 2, 4, _C, _K)
    Wz = jnp.einsum('op,prck->ckro', Wp, Wc4).reshape(_C, _K * _RO)
    b2r = (jnp.einsum('op,pr->ro', Wp, bc.reshape(Cin + 2, 4))
           + bp[None, :]).reshape(1, _RO)
    b2c = b2r.reshape(_RO, 1)

    z2 = pl.pallas_call(
        _zf_body,
        grid=(B,),
        in_specs=[
            pl.BlockSpec((1, _S, _C), lambda b: (b, 0, 0)),
            pl.BlockSpec((_C, _K * _RO), lambda b: (0, 0)),
        ],
        out_specs=pl.BlockSpec((1, _S, _K * _RO), lambda b: (b, 0, 0)),
        out_shape=jax.ShapeDtypeStruct((B, _S, _K * _RO), f32),
    )(xs, Wz)

    scvs = []
    for b in range(B):
        ids_b = pl.pallas_call(
            _score_body,
            grid=(1, t2),
            in_specs=[
                pl.BlockSpec((1, Cin, 2 * _R, W),
                             lambda bb, i: (bb, 0, i + _T1, 0)),
                pl.BlockSpec((1, 4, _S, 98), lambda bb, i: (bb, 0, 0, 0)),
            ],
            out_specs=pl.BlockSpec((1, _K, 1, _R, 112),
                                   lambda bb, i: (bb, 0, i, 0, 0)),
            out_shape=jax.ShapeDtypeStruct((1, _K, t2, _R, 112), jnp.int32),
        )(x[b:b + 1], xsijT[b:b + 1])
        zrow_b = z2[b].reshape(_S * _K, _RO)
        sc_b = _sc_gather(zrow_b, ids_b.reshape(_K * nsc), nsc)
        scvs.append(sc_b.reshape(1, t2, _TN, _RO))

    out = None
    for b in range(B):
        in_specs = [
            pl.BlockSpec((1, Cin, 2 * _R, W), lambda bb, i: (bb, 0, i, 0)),
            pl.BlockSpec((1, 4, _S, 98), lambda bb, i: (bb, 0, 0, 0)),
            pl.BlockSpec((1, _S, _K * _RO), lambda bb, i: (bb, 0, 0)),
            pl.BlockSpec((_RO, 1), lambda bb, i: (0, 0)),
        ]
        args = [x[b:b + 1], xsijT[b:b + 1], z2[b:b + 1], b2c]
        alias = {}
        if out is not None:
            in_specs.append(pl.BlockSpec(memory_space=pl.ANY))
            args.append(out)
            alias = {4: 0}
        out = pl.pallas_call(
            _fused_body,
            grid=(1, _T1),
            in_specs=in_specs,
            out_specs=pl.BlockSpec((1, 96, 2 * _R, W),
                                   lambda bb, i, b_=b: (b_, 0, i, 0)),
            out_shape=jax.ShapeDtypeStruct((B, 96, H, W), f32),
            input_output_aliases=alias,
        )(*args)

    for b in range(B):
        out = pl.pallas_call(
            _inter_body,
            grid=(t2,),
            in_specs=[
                pl.BlockSpec((1, 1, _TN, _RO), lambda i: (0, i, 0, 0)),
                pl.BlockSpec((1, _RO), lambda i: (0, 0)),
                pl.BlockSpec(memory_space=pl.ANY),
            ],
            out_specs=pl.BlockSpec((1, 96, 2 * _R, W),
                                   lambda i, b_=b: (b_, 0, i + _T1, 0)),
            out_shape=jax.ShapeDtypeStruct((B, 96, H, W), f32),
            input_output_aliases={2: 0},
        )(scvs[b], b2r, out)
    return out
